# baseline (speedup 1.0000x reference)
import functools

import jax
import jax.numpy as jnp
import numpy as np
from jax.experimental import pallas as pl
from jax.experimental.pallas import tpu as pltpu

N = 10000
D = 256
ROW_BLK = 1000


def _mm_kernel(x_ref, w_ref, o_ref):
    o_ref[...] = jnp.dot(x_ref[...], w_ref[...],
                         preferred_element_type=jnp.float32)


def _matmul(x, w):
    n, k = x.shape
    m = w.shape[1]
    grid = (n // ROW_BLK,)
    return pl.pallas_call(
        _mm_kernel,
        grid=grid,
        in_specs=[
            pl.BlockSpec((ROW_BLK, k), lambda i: (i, 0)),
            pl.BlockSpec((k, m), lambda i: (0, 0)),
        ],
        out_specs=pl.BlockSpec((ROW_BLK, m), lambda i: (i, 0)),
        out_shape=jax.ShapeDtypeStruct((n, m), jnp.float32),
    )(x, w)


def _readout_kernel(h_ref, w_ref, b_ref, o_ref):
    z = jnp.dot(h_ref[...], w_ref[...], preferred_element_type=jnp.float32)
    z = z + b_ref[...]
    z = z - jnp.max(z, axis=1, keepdims=True)
    e = jnp.exp(z)
    o_ref[...] = e / jnp.sum(e, axis=1, keepdims=True)


def _readout(h, w, b):
    n, k = h.shape
    m = w.shape[1]
    return pl.pallas_call(
        _readout_kernel,
        grid=(n // ROW_BLK,),
        in_specs=[
            pl.BlockSpec((ROW_BLK, k), lambda i: (i, 0)),
            pl.BlockSpec((k, m), lambda i: (0, 0)),
            pl.BlockSpec((1, m), lambda i: (0, 0)),
        ],
        out_specs=pl.BlockSpec((ROW_BLK, m), lambda i: (i, 0)),
        out_shape=jax.ShapeDtypeStruct((n, m), jnp.float32),
    )(h, w, b.reshape(1, m))


def _edge_phase(xl, xr, src, dst, att, b):
    n = xl.shape[0]
    h = jax.nn.leaky_relu(xl[src] + xr[dst], negative_slope=0.2)
    logits = h @ att
    m = jax.ops.segment_max(logits, dst, num_segments=n)
    p = jnp.exp(logits - m[dst])
    denom = jax.ops.segment_sum(p, dst, num_segments=n)
    alpha = p / denom[dst]
    out = jax.ops.segment_sum(alpha[:, None] * xl[src], dst, num_segments=n)
    return out + b


def kernel(x, edge_index, Wl1, Wr1, att1, b1, Wl2, Wr2, att2, b2, Wro, bro):
    n = x.shape[0]
    loops = jnp.arange(n, dtype=edge_index.dtype)
    src = jnp.concatenate([edge_index[0], loops])
    dst = jnp.concatenate([edge_index[1], loops])

    w1 = jnp.concatenate([Wl1, Wr1], axis=1)
    z1 = _matmul(x, w1)
    h = jax.nn.relu(_edge_phase(z1[:, :D], z1[:, D:], src, dst, att1, b1))

    w2 = jnp.concatenate([Wl2, Wr2], axis=1)
    z2 = _matmul(h, w2)
    h2 = jax.nn.relu(_edge_phase(z2[:, :D], z2[:, D:], src, dst, att2, b2))

    return _readout(h2, Wro, bro)


# speedup vs baseline: 1.4913x; 1.4913x over previous
import functools

import jax
import jax.numpy as jnp
from jax import lax
from jax.experimental import pallas as pl
from jax.experimental.pallas import tpu as pltpu
from jax.experimental.pallas import tpu_sc as plsc

N = 10000
D = 256
E = 160000
ET = E + N
NC, NS, L = 2, 16, 16
NW = NC * NS
NPW = -(-N // NW)
RPAL = 328
RP_PAD = 10016
ROW_BLK = 1000
NEG = -3.0e38


def _mm2_kernel(x_ref, wl_ref, wr_ref, ol_ref, or_ref):
    ol_ref[...] = jnp.dot(x_ref[...], wl_ref[...],
                          preferred_element_type=jnp.float32)
    or_ref[...] = jnp.dot(x_ref[...], wr_ref[...],
                          preferred_element_type=jnp.float32)


def _matmul2(x, wl, wr):
    n, k = x.shape
    m = wl.shape[1]
    return pl.pallas_call(
        _mm2_kernel,
        grid=(n // ROW_BLK,),
        in_specs=[
            pl.BlockSpec((ROW_BLK, k), lambda i: (i, 0)),
            pl.BlockSpec((k, m), lambda i: (0, 0)),
            pl.BlockSpec((k, m), lambda i: (0, 0)),
        ],
        out_specs=[
            pl.BlockSpec((ROW_BLK, m), lambda i: (i, 0)),
            pl.BlockSpec((ROW_BLK, m), lambda i: (i, 0)),
        ],
        out_shape=[
            jax.ShapeDtypeStruct((n, m), jnp.float32),
            jax.ShapeDtypeStruct((n, m), jnp.float32),
        ],
    )(x, wl, wr)


def _readout_kernel(h_ref, w_ref, b_ref, o_ref):
    z = jnp.dot(h_ref[...], w_ref[...], preferred_element_type=jnp.float32)
    z = z + b_ref[...]
    z = z - jnp.max(z, axis=1, keepdims=True)
    e = jnp.exp(z)
    o_ref[...] = e / jnp.sum(e, axis=1, keepdims=True)


def _readout(h, w, b):
    n, k = h.shape
    m = w.shape[1]
    return pl.pallas_call(
        _readout_kernel,
        grid=(n // ROW_BLK,),
        in_specs=[
            pl.BlockSpec((ROW_BLK, k), lambda i: (i, 0)),
            pl.BlockSpec((k, m), lambda i: (0, 0)),
            pl.BlockSpec((1, m), lambda i: (0, 0)),
        ],
        out_specs=pl.BlockSpec((ROW_BLK, m), lambda i: (i, 0)),
        out_shape=jax.ShapeDtypeStruct((n, m), jnp.float32),
    )(h, w, b.reshape(1, m))


def _sc_scalar(vec_ref, off):
    idx = jnp.broadcast_to(off, (L,)).astype(jnp.int32)
    return jnp.max(plsc.load_gather(vec_ref, (idx,)))


def _gat_edge_sc(xl, xr, srcs, rowptr, att_arr, bias):
    mesh = plsc.VectorSubcoreMesh(core_axis_name="c", subcore_axis_name="s")

    @functools.partial(
        pl.kernel,
        out_type=jax.ShapeDtypeStruct((N, D), jnp.float32),
        mesh=mesh,
        compiler_params=pltpu.CompilerParams(needs_layout_passes=False),
        scratch_types=[
            pltpu.VMEM((L,), jnp.int32),
            pltpu.VMEM((L, D), jnp.float32),
            pltpu.VMEM((RPAL,), jnp.int32),
            pltpu.VMEM((D,), jnp.float32),
            pltpu.VMEM((D,), jnp.float32),
            pltpu.VMEM((D,), jnp.float32),
            pltpu.VMEM((D,), jnp.float32),
            pltpu.VMEM((L,), jnp.float32),
            pltpu.SemaphoreType.DMA,
        ],
    )
    def k(xl_h, xr_h, srcs_h, rp_h, att_h, b_h, out_h,
          sidx_v, rows_v, rp_v, att_v, b_v, xr_v, acc_v, p_v, sem):
        wid = lax.axis_index("s") * NC + lax.axis_index("c")
        v0 = wid * NPW
        v1 = jnp.minimum(v0 + NPW, N)
        a0 = (v0 // 8) * 8
        pltpu.sync_copy(rp_h.at[pl.ds(pl.multiple_of(a0, 8), RPAL)], rp_v)
        pltpu.sync_copy(att_h, att_v)
        pltpu.sync_copy(b_h, b_v)
        li = lax.broadcasted_iota(jnp.int32, (L,), 0)

        def node_body(v, _carry):
            off = v - a0
            e0 = _sc_scalar(rp_v, off)
            e1 = _sc_scalar(rp_v, off + 1)
            pltpu.sync_copy(xr_h.at[v], xr_v)
            for jc in range(16):
                acc_v[pl.ds(jc * L, L)] = jnp.zeros((L,), jnp.float32)
            c0 = e0 // 16
            nch = (e1 + 15) // 16 - c0

            def chunk_body(c, car):
                m16, d16 = car
                eb = (c0 + c) * 16
                pltpu.sync_copy(srcs_h.at[pl.ds(pl.multiple_of(eb, 8), L)],
                                sidx_v)
                pltpu.async_copy(xl_h.at[sidx_v], rows_v, sem).wait()
                mask = ((eb + li) >= e0) & ((eb + li) < e1)
                def row_logit(i, lg):
                    ri = jnp.broadcast_to(i, (L,)).astype(jnp.int32)
                    racc = jnp.zeros((L,), jnp.float32)
                    for jc in range(16):
                        a = plsc.load_gather(rows_v, (ri, jc * L + li))
                        s = a + xr_v[pl.ds(jc * L, L)]
                        s = jnp.maximum(s, 0.2 * s)
                        racc = racc + s * att_v[pl.ds(jc * L, L)]
                    return jnp.where(li == i, jnp.sum(racc), lg)

                lg = lax.fori_loop(0, 16, row_logit,
                                   jnp.full((L,), NEG, jnp.float32))
                lg = jnp.where(mask, lg, NEG)
                mn16 = jnp.maximum(m16, jnp.broadcast_to(jnp.max(lg), (L,)))
                scale = jnp.exp(m16 - mn16)
                p16 = jnp.where(mask, jnp.exp(lg - mn16), 0.0)
                d16n = d16 * scale + p16
                p_v[...] = p16

                for jc in range(16):
                    sl = pl.ds(jc * L, L)
                    acc_v[sl] = acc_v[sl] * scale
                def row_acc2(i, _):
                    ri = jnp.broadcast_to(i, (L,)).astype(jnp.int32)
                    pb = plsc.load_gather(p_v, (ri,))
                    for jc in range(16):
                        a = plsc.load_gather(rows_v, (ri, jc * L + li))
                        sl = pl.ds(jc * L, L)
                        acc_v[sl] = acc_v[sl] + pb * a
                    return 0
                lax.fori_loop(0, 16, row_acc2, 0)
                return (mn16, d16n)

            m16, d16 = lax.fori_loop(
                0, nch, chunk_body,
                (jnp.full((L,), NEG, jnp.float32),
                 jnp.zeros((L,), jnp.float32)))
            inv = 1.0 / jnp.broadcast_to(jnp.sum(d16), (L,))
            for jc in range(16):
                sl = pl.ds(jc * L, L)
                acc_v[sl] = jnp.maximum(acc_v[sl] * inv + b_v[sl], 0.0)
            pltpu.sync_copy(acc_v, out_h.at[v])
            return 0

        lax.fori_loop(v0, v1, node_body, 0)

    return k(xl, xr, srcs, rowptr, att_arr, bias)


def kernel(x, edge_index, Wl1, Wr1, att1, b1, Wl2, Wr2, att2, b2, Wro, bro):
    loops = jnp.arange(N, dtype=edge_index.dtype)
    src = jnp.concatenate([edge_index[0], loops])
    dst = jnp.concatenate([edge_index[1], loops])
    dsts, srcs = lax.sort([dst, src], num_keys=1)
    rowptr = jnp.searchsorted(dsts, jnp.arange(N + 1, dtype=jnp.int32)
                              ).astype(jnp.int32)
    rowptr = jnp.concatenate(
        [rowptr, jnp.full((RP_PAD - (N + 1),), ET, jnp.int32)])

    xl1, xr1 = _matmul2(x, Wl1, Wr1)
    h = _gat_edge_sc(xl1, xr1, srcs, rowptr, att1, b1)
    xl2, xr2 = _matmul2(h, Wl2, Wr2)
    h2 = _gat_edge_sc(xl2, xr2, srcs, rowptr, att2, b2)
    return _readout(h2, Wro, bro)


# speedup vs baseline: 1.5329x; 1.0279x over previous
import functools

import jax
import jax.numpy as jnp
from jax import lax
from jax.experimental import pallas as pl
from jax.experimental.pallas import tpu as pltpu
from jax.experimental.pallas import tpu_sc as plsc

N = 10000
D = 256
E = 160000
ET = E + N
NC, NS, L = 2, 16, 16
NW = NC * NS
NP = 10240
NPW = 320
RPAL = 328
RP_PAD = NW * NPW + RPAL
ROW_BLK = 1024
NEG = -3.0e38


def _mm2_kernel(x_ref, wl_ref, wr_ref, ol_ref, or_ref):
    ol_ref[...] = jnp.dot(x_ref[...], wl_ref[...],
                          preferred_element_type=jnp.float32)
    or_ref[...] = jnp.dot(x_ref[...], wr_ref[...],
                          preferred_element_type=jnp.float32)


def _matmul2(x, wl, wr):
    n, k = x.shape
    m = wl.shape[1]
    return pl.pallas_call(
        _mm2_kernel,
        grid=(n // ROW_BLK,),
        in_specs=[
            pl.BlockSpec((ROW_BLK, k), lambda i: (i, 0)),
            pl.BlockSpec((k, m), lambda i: (0, 0)),
            pl.BlockSpec((k, m), lambda i: (0, 0)),
        ],
        out_specs=[
            pl.BlockSpec((ROW_BLK, m), lambda i: (i, 0)),
            pl.BlockSpec((ROW_BLK, m), lambda i: (i, 0)),
        ],
        out_shape=[
            jax.ShapeDtypeStruct((n, m), jnp.float32),
            jax.ShapeDtypeStruct((n, m), jnp.float32),
        ],
    )(x, wl, wr)


def _readout_kernel(h_ref, w_ref, b_ref, o_ref):
    z = jnp.dot(h_ref[...], w_ref[...], preferred_element_type=jnp.float32)
    z = z + b_ref[...]
    z = z - jnp.max(z, axis=1, keepdims=True)
    e = jnp.exp(z)
    o_ref[...] = e / jnp.sum(e, axis=1, keepdims=True)


def _readout(h, w, b):
    n, k = h.shape
    m = w.shape[1]
    return pl.pallas_call(
        _readout_kernel,
        grid=(n // ROW_BLK,),
        in_specs=[
            pl.BlockSpec((ROW_BLK, k), lambda i: (i, 0)),
            pl.BlockSpec((k, m), lambda i: (0, 0)),
            pl.BlockSpec((1, m), lambda i: (0, 0)),
        ],
        out_specs=pl.BlockSpec((ROW_BLK, m), lambda i: (i, 0)),
        out_shape=jax.ShapeDtypeStruct((n, m), jnp.float32),
    )(h, w, b.reshape(1, m))


def _sc_scalar(vec_ref, off):
    idx = jnp.broadcast_to(off, (L,)).astype(jnp.int32)
    return jnp.max(plsc.load_gather(vec_ref, (idx,)))


def _gat_edge_sc(xl, xr, srcs, rowptr, att_arr, bias):
    mesh = plsc.VectorSubcoreMesh(core_axis_name="c", subcore_axis_name="s")

    @functools.partial(
        pl.kernel,
        out_type=jax.ShapeDtypeStruct((NP, D), jnp.float32),
        mesh=mesh,
        compiler_params=pltpu.CompilerParams(needs_layout_passes=False),
        scratch_types=[
            pltpu.VMEM((L,), jnp.int32),
            pltpu.VMEM((L, D), jnp.float32),
            pltpu.VMEM((RPAL,), jnp.int32),
            pltpu.VMEM((D,), jnp.float32),
            pltpu.VMEM((D,), jnp.float32),
            pltpu.VMEM((NPW, D), jnp.float32),
            pltpu.VMEM((D,), jnp.float32),
            pltpu.VMEM((D,), jnp.float32),
            pltpu.VMEM((16, D), jnp.float32),
            pltpu.VMEM((L,), jnp.float32),
            pltpu.SemaphoreType.DMA,
        ],
    )
    def k(xl_h, xr_h, srcs_h, rp_h, att_h, b_h, out_h,
          sidx_v, rows_v, rp_v, att_v, b_v, xrs_v, xr_v, acc_v, o_v, p_v,
          sem):
        wid = lax.axis_index("s") * NC + lax.axis_index("c")
        v0 = pl.multiple_of(wid * NPW, 8)
        pltpu.sync_copy(rp_h.at[pl.ds(v0, RPAL)], rp_v)
        pltpu.sync_copy(att_h, att_v)
        pltpu.sync_copy(b_h, b_v)
        pltpu.sync_copy(xr_h.at[pl.ds(v0, NPW)], xrs_v)
        li = lax.broadcasted_iota(jnp.int32, (L,), 0)

        def node_body(nl, _carry):
            e0 = _sc_scalar(rp_v, nl)
            e1 = _sc_scalar(rp_v, nl + 1)
            nli = jnp.broadcast_to(nl, (L,)).astype(jnp.int32)
            for jc in range(16):
                sl = pl.ds(jc * L, L)
                xr_v[sl] = plsc.load_gather(xrs_v, (nli, jc * L + li))
                acc_v[sl] = jnp.zeros((L,), jnp.float32)
            c0 = e0 // 16
            nch = (e1 + 15) // 16 - c0

            def chunk_body(c, car):
                m16, d16 = car
                eb = (c0 + c) * 16
                pltpu.sync_copy(srcs_h.at[pl.ds(pl.multiple_of(eb, 8), L)],
                                sidx_v)
                pltpu.async_copy(xl_h.at[sidx_v], rows_v, sem).wait()
                mask = ((eb + li) >= e0) & ((eb + li) < e1)

                def row_logit(i, lg):
                    ri = jnp.broadcast_to(i, (L,)).astype(jnp.int32)
                    racc = jnp.zeros((L,), jnp.float32)
                    for jc in range(16):
                        a = plsc.load_gather(rows_v, (ri, jc * L + li))
                        s = a + xr_v[pl.ds(jc * L, L)]
                        s = jnp.maximum(s, 0.2 * s)
                        racc = racc + s * att_v[pl.ds(jc * L, L)]
                    return jnp.where(li == i, jnp.sum(racc), lg)

                lg = lax.fori_loop(0, 16, row_logit,
                                   jnp.full((L,), NEG, jnp.float32))
                lg = jnp.where(mask, lg, NEG)
                mn16 = jnp.maximum(m16, jnp.broadcast_to(jnp.max(lg), (L,)))
                scale = jnp.exp(m16 - mn16)
                p16 = jnp.where(mask, jnp.exp(lg - mn16), 0.0)
                d16n = d16 * scale + p16
                p_v[...] = p16

                for jc in range(16):
                    sl = pl.ds(jc * L, L)
                    acc_v[sl] = acc_v[sl] * scale

                def row_acc(i, _):
                    ri = jnp.broadcast_to(i, (L,)).astype(jnp.int32)
                    pb = plsc.load_gather(p_v, (ri,))
                    for jc in range(16):
                        a = plsc.load_gather(rows_v, (ri, jc * L + li))
                        sl = pl.ds(jc * L, L)
                        acc_v[sl] = acc_v[sl] + pb * a
                    return 0

                lax.fori_loop(0, 16, row_acc, 0)
                return (mn16, d16n)

            m16, d16 = lax.fori_loop(
                0, nch, chunk_body,
                (jnp.full((L,), NEG, jnp.float32),
                 jnp.zeros((L,), jnp.float32)))
            inv = 1.0 / jnp.broadcast_to(jnp.sum(d16), (L,))
            ri = jnp.broadcast_to(nl % 16, (L,)).astype(jnp.int32)
            for jc in range(16):
                sl = pl.ds(jc * L, L)
                plsc.store_scatter(
                    o_v, (ri, jc * L + li),
                    jnp.maximum(acc_v[sl] * inv + b_v[sl], 0.0))

            @pl.when(nl % 16 == 15)
            def _flush():
                vb = pl.multiple_of(v0 + nl - 15, 8)
                pltpu.sync_copy(o_v, out_h.at[pl.ds(vb, 16)])

            return 0

        lax.fori_loop(0, NPW, node_body, 0)

    return k(xl, xr, srcs, rowptr, att_arr, bias)


def kernel(x, edge_index, Wl1, Wr1, att1, b1, Wl2, Wr2, att2, b2, Wro, bro):
    loops = jnp.arange(N, dtype=edge_index.dtype)
    src = jnp.concatenate([edge_index[0], loops])
    dst = jnp.concatenate([edge_index[1], loops])
    dsts, srcs = lax.sort([dst, src], num_keys=1)
    rowptr = jnp.searchsorted(dsts, jnp.arange(N + 1, dtype=jnp.int32)
                              ).astype(jnp.int32)
    rowptr = jnp.concatenate(
        [rowptr, jnp.full((RP_PAD - (N + 1),), ET, jnp.int32)])

    xp = jnp.pad(x, ((0, NP - N), (0, 0)))
    xl1, xr1 = _matmul2(xp, Wl1, Wr1)
    h = _gat_edge_sc(xl1, xr1, srcs, rowptr, att1, b1)
    xl2, xr2 = _matmul2(h, Wl2, Wr2)
    h2 = _gat_edge_sc(xl2, xr2, srcs, rowptr, att2, b2)
    return _readout(h2, Wro, bro)[:N]


# speedup vs baseline: 1.9513x; 1.2729x over previous
import functools

import jax
import jax.numpy as jnp
from jax import lax
from jax.experimental import pallas as pl
from jax.experimental.pallas import tpu as pltpu
from jax.experimental.pallas import tpu_sc as plsc

N = 10000
D = 256
E = 160000
ET = E + N
NC, NS, L = 2, 16, 16
NW = NC * NS
NP = 10240
NPW = 320
RPAL = 328
RP_PAD = NW * NPW + RPAL
SBLK = 8192
SP_PAD = ET + SBLK
ROW_BLK = 1024
NEG = -3.0e38


def _mm2_kernel(x_ref, wl_ref, wr_ref, ol_ref, or_ref):
    ol_ref[...] = jnp.dot(x_ref[...], wl_ref[...],
                          preferred_element_type=jnp.float32)
    or_ref[...] = jnp.dot(x_ref[...], wr_ref[...],
                          preferred_element_type=jnp.float32)


def _matmul2(x, wl, wr):
    n, k = x.shape
    m = wl.shape[1]
    return pl.pallas_call(
        _mm2_kernel,
        grid=(n // ROW_BLK,),
        in_specs=[
            pl.BlockSpec((ROW_BLK, k), lambda i: (i, 0)),
            pl.BlockSpec((k, m), lambda i: (0, 0)),
            pl.BlockSpec((k, m), lambda i: (0, 0)),
        ],
        out_specs=[
            pl.BlockSpec((ROW_BLK, m), lambda i: (i, 0)),
            pl.BlockSpec((ROW_BLK, m), lambda i: (i, 0)),
        ],
        out_shape=[
            jax.ShapeDtypeStruct((n, m), jnp.float32),
            jax.ShapeDtypeStruct((n, m), jnp.float32),
        ],
    )(x, wl, wr)


def _readout_kernel(h_ref, w_ref, b_ref, o_ref):
    z = jnp.dot(h_ref[...], w_ref[...], preferred_element_type=jnp.float32)
    z = z + b_ref[...]
    z = z - jnp.max(z, axis=1, keepdims=True)
    e = jnp.exp(z)
    o_ref[...] = e / jnp.sum(e, axis=1, keepdims=True)


def _readout(h, w, b):
    n, k = h.shape
    m = w.shape[1]
    return pl.pallas_call(
        _readout_kernel,
        grid=(n // ROW_BLK,),
        in_specs=[
            pl.BlockSpec((ROW_BLK, k), lambda i: (i, 0)),
            pl.BlockSpec((k, m), lambda i: (0, 0)),
            pl.BlockSpec((1, m), lambda i: (0, 0)),
        ],
        out_specs=pl.BlockSpec((ROW_BLK, m), lambda i: (i, 0)),
        out_shape=jax.ShapeDtypeStruct((n, m), jnp.float32),
    )(h, w, b.reshape(1, m))


def _sc_scalar(vec_ref, off):
    idx = jnp.broadcast_to(off, (L,)).astype(jnp.int32)
    return jnp.max(plsc.load_gather(vec_ref, (idx,)))


def _gat_edge_sc(xl, xr, srcs, rowptr, att_arr, bias):
    mesh = plsc.VectorSubcoreMesh(core_axis_name="c", subcore_axis_name="s")

    @functools.partial(
        pl.kernel,
        out_type=jax.ShapeDtypeStruct((NP, D), jnp.float32),
        mesh=mesh,
        compiler_params=pltpu.CompilerParams(needs_layout_passes=False),
        scratch_types=[
            pltpu.VMEM((L,), jnp.int32),
            pltpu.VMEM((L, D), jnp.float32),
            pltpu.VMEM((RPAL,), jnp.int32),
            pltpu.VMEM((D,), jnp.float32),
            pltpu.VMEM((D,), jnp.float32),
            pltpu.VMEM((NPW, D), jnp.float32),
            pltpu.VMEM((D,), jnp.float32),
            pltpu.VMEM((D,), jnp.float32),
            pltpu.VMEM((16, D), jnp.float32),
            pltpu.VMEM((L,), jnp.float32),
            pltpu.SemaphoreType.DMA,
        ],
    )
    def k(xl_h, xr_h, srcs_h, rp_h, att_h, b_h, out_h,
          sidx_v, rows_v, rp_v, att_v, b_v, xrs_v, xr_v, acc_v, o_v, p_v,
          sem0):
        wid = lax.axis_index("s") * NC + lax.axis_index("c")
        v0 = pl.multiple_of(wid * NPW, 8)
        pltpu.sync_copy(rp_h.at[pl.ds(v0, RPAL)], rp_v)
        pltpu.sync_copy(att_h, att_v)
        pltpu.sync_copy(b_h, b_v)
        pltpu.sync_copy(xr_h.at[pl.ds(v0, NPW)], xrs_v)
        li = lax.broadcasted_iota(jnp.int32, (L,), 0)

        def node_body(nl, _carry):
            e0 = _sc_scalar(rp_v, nl)
            e1 = _sc_scalar(rp_v, nl + 1)
            nli = jnp.broadcast_to(nl, (L,)).astype(jnp.int32)
            for jc in range(16):
                sl = pl.ds(jc * L, L)
                xr_v[sl] = plsc.load_gather(xrs_v, (nli, jc * L + li))
                acc_v[sl] = jnp.zeros((L,), jnp.float32)
            c0 = e0 // 16
            nch = (e1 + 15) // 16 - c0

            def chunk_body(c, car):
                m16, d16 = car
                eb = (c0 + c) * 16
                pltpu.sync_copy(srcs_h.at[pl.ds(pl.multiple_of(eb, 8), L)],
                                sidx_v)
                pltpu.async_copy(xl_h.at[sidx_v], rows_v, sem0).wait()
                mask = ((eb + li) >= e0) & ((eb + li) < e1)
                lo = jnp.maximum(eb, e0) - eb
                hi = jnp.minimum(eb + 16, e1) - eb

                def row_logit(i, lg):
                    ri = jnp.broadcast_to(i, (L,)).astype(jnp.int32)
                    racc = jnp.zeros((L,), jnp.float32)
                    for jc in range(16):
                        a = plsc.load_gather(rows_v, (ri, jc * L + li))
                        s = a + xr_v[pl.ds(jc * L, L)]
                        s = jnp.maximum(s, 0.2 * s)
                        racc = racc + s * att_v[pl.ds(jc * L, L)]
                    return jnp.where(li == i, jnp.sum(racc), lg)

                lg = lax.fori_loop(lo, hi, row_logit,
                                   jnp.full((L,), NEG, jnp.float32))
                lg = jnp.where(mask, lg, NEG)
                mn16 = jnp.maximum(m16, jnp.broadcast_to(jnp.max(lg), (L,)))
                scale = jnp.exp(m16 - mn16)
                p16 = jnp.where(mask, jnp.exp(lg - mn16), 0.0)
                d16n = d16 * scale + p16
                p_v[...] = p16

                @pl.when(jnp.max(mn16) > jnp.max(m16))
                def _():
                    for jc in range(16):
                        sl = pl.ds(jc * L, L)
                        acc_v[sl] = acc_v[sl] * scale

                def row_acc(i, _):
                    ri = jnp.broadcast_to(i, (L,)).astype(jnp.int32)
                    pb = plsc.load_gather(p_v, (ri,))
                    for jc in range(16):
                        a = plsc.load_gather(rows_v, (ri, jc * L + li))
                        sl = pl.ds(jc * L, L)
                        acc_v[sl] = acc_v[sl] + pb * a
                    return 0

                lax.fori_loop(lo, hi, row_acc, 0)
                return (mn16, d16n)

            m16, d16 = lax.fori_loop(
                0, nch, chunk_body,
                (jnp.full((L,), NEG, jnp.float32),
                 jnp.zeros((L,), jnp.float32)))
            inv = 1.0 / jnp.broadcast_to(jnp.sum(d16), (L,))
            ri = jnp.broadcast_to(nl % 16, (L,)).astype(jnp.int32)
            for jc in range(16):
                sl = pl.ds(jc * L, L)
                plsc.store_scatter(
                    o_v, (ri, jc * L + li),
                    jnp.maximum(acc_v[sl] * inv + b_v[sl], 0.0))

            @pl.when(nl % 16 == 15)
            def _flush():
                vb = pl.multiple_of(v0 + nl - 15, 8)
                pltpu.sync_copy(o_v, out_h.at[pl.ds(vb, 16)])

            return 0

        lax.fori_loop(0, NPW, node_body, 0)

    return k(xl, xr, srcs, rowptr, att_arr, bias)


def kernel(x, edge_index, Wl1, Wr1, att1, b1, Wl2, Wr2, att2, b2, Wro, bro):
    loops = jnp.arange(N, dtype=edge_index.dtype)
    src = jnp.concatenate([edge_index[0], loops])
    dst = jnp.concatenate([edge_index[1], loops])
    dsts, srcs = lax.sort([dst, src], num_keys=1)
    srcs = jnp.pad(srcs, (0, SP_PAD - ET))
    rowptr = jnp.searchsorted(dsts, jnp.arange(N + 1, dtype=jnp.int32)
                              ).astype(jnp.int32)
    rowptr = jnp.concatenate(
        [rowptr, jnp.full((RP_PAD - (N + 1),), ET, jnp.int32)])

    xp = jnp.pad(x, ((0, NP - N), (0, 0)))
    xl1, xr1 = _matmul2(xp, Wl1, Wr1)
    h = _gat_edge_sc(xl1, xr1, srcs, rowptr, att1, b1)
    xl2, xr2 = _matmul2(h, Wl2, Wr2)
    h2 = _gat_edge_sc(xl2, xr2, srcs, rowptr, att2, b2)
    return _readout(h2, Wro, bro)[:N]


# speedup vs baseline: 2.0434x; 1.0472x over previous
import functools

import jax
import jax.numpy as jnp
from jax import lax
from jax.experimental import pallas as pl
from jax.experimental.pallas import tpu as pltpu
from jax.experimental.pallas import tpu_sc as plsc

N = 10000
D = 256
E = 160000
ET = E + N
NC, NS, L = 2, 16, 16
NW = NC * NS
NP = 10240
NPW = 320
CH = 32
RPAL = 328
RP_PAD = NW * NPW + RPAL
SBLK = 8192
SP_PAD = ET + SBLK
ROW_BLK = 1024
NEG = -3.0e38


def _mm2_kernel(x_ref, wl_ref, wr_ref, ol_ref, or_ref):
    ol_ref[...] = jnp.dot(x_ref[...], wl_ref[...],
                          preferred_element_type=jnp.float32)
    or_ref[...] = jnp.dot(x_ref[...], wr_ref[...],
                          preferred_element_type=jnp.float32)


def _matmul2(x, wl, wr):
    n, k = x.shape
    m = wl.shape[1]
    return pl.pallas_call(
        _mm2_kernel,
        grid=(n // ROW_BLK,),
        in_specs=[
            pl.BlockSpec((ROW_BLK, k), lambda i: (i, 0)),
            pl.BlockSpec((k, m), lambda i: (0, 0)),
            pl.BlockSpec((k, m), lambda i: (0, 0)),
        ],
        out_specs=[
            pl.BlockSpec((ROW_BLK, m), lambda i: (i, 0)),
            pl.BlockSpec((ROW_BLK, m), lambda i: (i, 0)),
        ],
        out_shape=[
            jax.ShapeDtypeStruct((n, m), jnp.float32),
            jax.ShapeDtypeStruct((n, m), jnp.float32),
        ],
    )(x, wl, wr)


def _readout_kernel(h_ref, w_ref, b_ref, o_ref):
    z = jnp.dot(h_ref[...], w_ref[...], preferred_element_type=jnp.float32)
    z = z + b_ref[...]
    z = z - jnp.max(z, axis=1, keepdims=True)
    e = jnp.exp(z)
    o_ref[...] = e / jnp.sum(e, axis=1, keepdims=True)


def _readout(h, w, b):
    n, k = h.shape
    m = w.shape[1]
    return pl.pallas_call(
        _readout_kernel,
        grid=(n // ROW_BLK,),
        in_specs=[
            pl.BlockSpec((ROW_BLK, k), lambda i: (i, 0)),
            pl.BlockSpec((k, m), lambda i: (0, 0)),
            pl.BlockSpec((1, m), lambda i: (0, 0)),
        ],
        out_specs=pl.BlockSpec((ROW_BLK, m), lambda i: (i, 0)),
        out_shape=jax.ShapeDtypeStruct((n, m), jnp.float32),
    )(h, w, b.reshape(1, m))


def _sc_scalar(vec_ref, off):
    idx = jnp.broadcast_to(off, (L,)).astype(jnp.int32)
    return jnp.max(plsc.load_gather(vec_ref, (idx,)))


def _gat_edge_sc(xl, xr, srcs, rowptr, att_arr, bias):
    mesh = plsc.VectorSubcoreMesh(core_axis_name="c", subcore_axis_name="s")

    @functools.partial(
        pl.kernel,
        out_type=jax.ShapeDtypeStruct((NP, D), jnp.float32),
        mesh=mesh,
        compiler_params=pltpu.CompilerParams(needs_layout_passes=False),
        scratch_types=[
            pltpu.VMEM((CH,), jnp.int32),
            pltpu.VMEM((CH, D), jnp.float32),
            pltpu.VMEM((RPAL,), jnp.int32),
            pltpu.VMEM((D,), jnp.float32),
            pltpu.VMEM((D,), jnp.float32),
            pltpu.VMEM((NPW, D), jnp.float32),
            pltpu.VMEM((D,), jnp.float32),
            pltpu.VMEM((D,), jnp.float32),
            pltpu.VMEM((16, D), jnp.float32),
            pltpu.VMEM((CH,), jnp.float32),
            pltpu.SemaphoreType.DMA,
        ],
    )
    def k(xl_h, xr_h, srcs_h, rp_h, att_h, b_h, out_h,
          sidx_v, rows_v, rp_v, att_v, b_v, xrs_v, xr_v, acc_v, o_v, p_v,
          sem0):
        wid = lax.axis_index("s") * NC + lax.axis_index("c")
        v0 = pl.multiple_of(wid * NPW, 8)
        pltpu.sync_copy(rp_h.at[pl.ds(v0, RPAL)], rp_v)
        pltpu.sync_copy(att_h, att_v)
        pltpu.sync_copy(b_h, b_v)
        pltpu.sync_copy(xr_h.at[pl.ds(v0, NPW)], xrs_v)
        li = lax.broadcasted_iota(jnp.int32, (L,), 0)

        def node_body(nl, _carry):
            e0 = _sc_scalar(rp_v, nl)
            e1 = _sc_scalar(rp_v, nl + 1)
            nli = jnp.broadcast_to(nl, (L,)).astype(jnp.int32)
            for jc in range(16):
                sl = pl.ds(jc * L, L)
                xr_v[sl] = plsc.load_gather(xrs_v, (nli, jc * L + li))
                acc_v[sl] = jnp.zeros((L,), jnp.float32)
            c0 = e0 // CH
            nch = jnp.where(e1 > e0, (e1 + CH - 1) // CH - c0, 0)

            def chunk_body(c, car):
                m16, d16 = car
                eb = (c0 + c) * CH
                pltpu.sync_copy(srcs_h.at[pl.ds(pl.multiple_of(eb, 8), CH)],
                                sidx_v)
                pltpu.async_copy(xl_h.at[sidx_v], rows_v, sem0).wait()
                lo = jnp.maximum(eb, e0) - eb
                hi = jnp.minimum(eb + CH, e1) - eb

                def row_logit(i, lgs):
                    lg0, lg1 = lgs
                    ri = jnp.broadcast_to(i, (L,)).astype(jnp.int32)
                    racc = jnp.zeros((L,), jnp.float32)
                    for jc in range(16):
                        a = plsc.load_gather(rows_v, (ri, jc * L + li))
                        s = a + xr_v[pl.ds(jc * L, L)]
                        s = jnp.maximum(s, 0.2 * s)
                        racc = racc + s * att_v[pl.ds(jc * L, L)]
                    sc = jnp.sum(racc)
                    lg0 = jnp.where(li == i, sc, lg0)
                    lg1 = jnp.where(li == i - 16, sc, lg1)
                    return (lg0, lg1)

                lg0, lg1 = lax.fori_loop(
                    lo, hi, row_logit,
                    (jnp.full((L,), NEG, jnp.float32),
                     jnp.full((L,), NEG, jnp.float32)))
                mask0 = ((eb + li) >= e0) & ((eb + li) < e1)
                mask1 = ((eb + 16 + li) >= e0) & ((eb + 16 + li) < e1)
                lg0 = jnp.where(mask0, lg0, NEG)
                lg1 = jnp.where(mask1, lg1, NEG)
                mx = jnp.maximum(jnp.max(lg0), jnp.max(lg1))
                mn16 = jnp.maximum(m16, jnp.broadcast_to(mx, (L,)))
                scale = jnp.exp(m16 - mn16)
                p0 = jnp.where(mask0, jnp.exp(lg0 - mn16), 0.0)
                p1 = jnp.where(mask1, jnp.exp(lg1 - mn16), 0.0)
                d16n = d16 * scale + p0 + p1
                p_v[pl.ds(0, L)] = p0
                p_v[pl.ds(16, L)] = p1

                @pl.when(jnp.max(mn16) > jnp.max(m16))
                def _():
                    for jc in range(16):
                        sl = pl.ds(jc * L, L)
                        acc_v[sl] = acc_v[sl] * scale

                def row_acc(i, _):
                    ri = jnp.broadcast_to(i, (L,)).astype(jnp.int32)
                    pb = plsc.load_gather(p_v, (ri,))
                    for jc in range(16):
                        a = plsc.load_gather(rows_v, (ri, jc * L + li))
                        sl = pl.ds(jc * L, L)
                        acc_v[sl] = acc_v[sl] + pb * a
                    return 0

                lax.fori_loop(lo, hi, row_acc, 0)
                return (mn16, d16n)

            m16, d16 = lax.fori_loop(
                0, nch, chunk_body,
                (jnp.full((L,), NEG, jnp.float32),
                 jnp.zeros((L,), jnp.float32)))
            inv = 1.0 / jnp.broadcast_to(jnp.sum(d16), (L,))
            ri = jnp.broadcast_to(nl % 16, (L,)).astype(jnp.int32)
            for jc in range(16):
                sl = pl.ds(jc * L, L)
                plsc.store_scatter(
                    o_v, (ri, jc * L + li),
                    jnp.maximum(acc_v[sl] * inv + b_v[sl], 0.0))

            @pl.when(nl % 16 == 15)
            def _flush():
                vb = pl.multiple_of(v0 + nl - 15, 8)
                pltpu.sync_copy(o_v, out_h.at[pl.ds(vb, 16)])

            return 0

        lax.fori_loop(0, NPW, node_body, 0)

    return k(xl, xr, srcs, rowptr, att_arr, bias)


def kernel(x, edge_index, Wl1, Wr1, att1, b1, Wl2, Wr2, att2, b2, Wro, bro):
    loops = jnp.arange(N, dtype=edge_index.dtype)
    src = jnp.concatenate([edge_index[0], loops])
    dst = jnp.concatenate([edge_index[1], loops])
    dsts, srcs = lax.sort([dst, src], num_keys=1)
    srcs = jnp.pad(srcs, (0, SP_PAD - ET))
    rowptr = jnp.searchsorted(dsts, jnp.arange(N + 1, dtype=jnp.int32)
                              ).astype(jnp.int32)
    rowptr = jnp.concatenate(
        [rowptr, jnp.full((RP_PAD - (N + 1),), ET, jnp.int32)])

    xp = jnp.pad(x, ((0, NP - N), (0, 0)))
    xl1, xr1 = _matmul2(xp, Wl1, Wr1)
    h = _gat_edge_sc(xl1, xr1, srcs, rowptr, att1, b1)
    xl2, xr2 = _matmul2(h, Wl2, Wr2)
    h2 = _gat_edge_sc(xl2, xr2, srcs, rowptr, att2, b2)
    return _readout(h2, Wro, bro)[:N]
